# Initial kernel scaffold; baseline (speedup 1.0000x reference)
import functools

import jax
import jax.numpy as jnp
from jax import lax
from jax.experimental import pallas as pl
from jax.experimental.pallas import tpu as pltpu
from jax.experimental.pallas import tpu_sc as plsc

_NUM_WORDS = 100000
_T = 32
_D = 64
_LANES = 16

_NC = 2
_NS = 16
_NW = _NC * _NS

_CH = 32
_G = 4


def _body(ids_hbm, toks_hbm, cnts_hbm, emb_hbm, idf_hbm, out_hbm,
          idf_v, ids_v, toks_v, cnts_v, emb_v, w_v, out_v, sem, gsem):
    qpw = ids_v.shape[0]
    wid = lax.axis_index("s") * _NC + lax.axis_index("c")
    base = wid * qpw

    pltpu.sync_copy(idf_hbm, idf_v)
    pltpu.sync_copy(ids_hbm.at[pl.ds(base, qpw)], ids_v)

    @pl.loop(0, qpw // _CH)
    def _chunk(c):
        idx = ids_v.at[pl.ds(c * _CH, _CH)]
        cp_t = pltpu.async_copy(toks_hbm.at[idx], toks_v, sem)
        cp_c = pltpu.async_copy(cnts_hbm.at[idx], cnts_v, sem)
        cp_t.wait()
        cp_c.wait()

        @pl.loop(0, _CH // _G)
        def _group(g):
            cps = [
                pltpu.async_copy(
                    emb_hbm.at[toks_v.at[g * _G + j]], emb_v.at[j], gsem)
                for j in range(_G)
            ]
            for cp in cps:
                cp.wait()

            for j in range(_G):
                q = g * _G + j
                t0 = toks_v[q, pl.ds(0, _LANES)]
                t1 = toks_v[q, pl.ds(_LANES, _LANES)]
                w0 = cnts_v[q, pl.ds(0, _LANES)] * plsc.load_gather(idf_v, [t0])
                w1 = cnts_v[q, pl.ds(_LANES, _LANES)] * plsc.load_gather(idf_v, [t1])
                w_v[pl.ds(0, _LANES)] = w0
                w_v[pl.ds(_LANES, _LANES)] = w1
                scale = 1.0 / jnp.maximum(jnp.sum(w0 + w1), 1e-6)

                accs = [jnp.zeros((_LANES,), jnp.float32) for _ in range(_D // _LANES)]
                for t in range(_T):
                    wt = plsc.load_gather(w_v, [jnp.full((_LANES,), t, jnp.int32)])
                    for s in range(_D // _LANES):
                        accs[s] = accs[s] + wt * emb_v[j, t, pl.ds(s * _LANES, _LANES)]
                for s in range(_D // _LANES):
                    out_v[q, pl.ds(s * _LANES, _LANES)] = accs[s] * scale

        pltpu.sync_copy(out_v, out_hbm.at[pl.ds(base + c * _CH, _CH)])


def kernel(entity_id, entity_tokens, entity_counts, word_embeds, idf):
    b, l = entity_id.shape
    q = b * l
    qpw = q // _NW
    flat_ids = entity_id.reshape(q).astype(jnp.int32)

    mesh = plsc.VectorSubcoreMesh(core_axis_name="c", subcore_axis_name="s")
    run = functools.partial(
        pl.kernel,
        out_type=jax.ShapeDtypeStruct((q, _D), jnp.float32),
        mesh=mesh,
        scratch_types=[
            pltpu.VMEM((_NUM_WORDS,), jnp.float32),
            pltpu.VMEM((qpw,), jnp.int32),
            pltpu.VMEM((_CH, _T), jnp.int32),
            pltpu.VMEM((_CH, _T), jnp.float32),
            pltpu.VMEM((_G, _T, _D), jnp.float32),
            pltpu.VMEM((_T,), jnp.float32),
            pltpu.VMEM((_CH, _D), jnp.float32),
            pltpu.SemaphoreType.DMA,
            pltpu.SemaphoreType.DMA,
        ],
    )(_body)
    out = run(flat_ids, entity_tokens, entity_counts, word_embeds, idf)
    return out.reshape(b, l, _D)


# speedup vs baseline: 25.7781x; 25.7781x over previous
import functools

import jax
import jax.numpy as jnp
from jax import lax
from jax.experimental import pallas as pl
from jax.experimental.pallas import tpu as pltpu
from jax.experimental.pallas import tpu_sc as plsc

_NUM_WORDS = 100000
_T = 32
_D = 64
_LANES = 16

_NC = 2
_NS = 16
_NW = _NC * _NS

_CH = 8


def _body(ids_hbm, toks_hbm, cnts_hbm, emb_hbm, idf_hbm, out_hbm,
          idf_v, ids_v, toks_v, cnts_v, emb_v, out_v, sem, gsem):
    qpw = ids_v.shape[0]
    wid = lax.axis_index("s") * _NC + lax.axis_index("c")
    base = wid * qpw

    pltpu.sync_copy(idf_hbm, idf_v)
    pltpu.sync_copy(ids_hbm.at[pl.ds(base, qpw)], ids_v)

    @pl.loop(0, qpw // _CH)
    def _chunk(c):
        idx = ids_v.at[pl.ds(c * _CH, _CH)]
        cp_t = pltpu.async_copy(toks_hbm.at[idx], toks_v, sem)
        cp_c = pltpu.async_copy(cnts_hbm.at[idx], cnts_v, sem)
        cp_t.wait()
        cp_c.wait()

        cps = [
            pltpu.async_copy(emb_hbm.at[toks_v.at[j]], emb_v.at[j], gsem)
            for j in range(_CH)
        ]
        for cp in cps:
            cp.wait()

        for j in range(_CH):
            t0 = toks_v[j, pl.ds(0, _LANES)]
            t1 = toks_v[j, pl.ds(_LANES, _LANES)]
            w0 = cnts_v[j, pl.ds(0, _LANES)] * plsc.load_gather(idf_v, [t0])
            w1 = cnts_v[j, pl.ds(_LANES, _LANES)] * plsc.load_gather(idf_v, [t1])
            denom = jnp.maximum(jnp.sum(w0 + w1), 1e-6)
            scale = 1.0 / jnp.broadcast_to(denom, (_LANES,))

            accs = [jnp.zeros((_LANES,), jnp.float32) for _ in range(_D // _LANES)]
            for half, wv in enumerate((w0, w1)):
                for tt in range(_LANES):
                    t = half * _LANES + tt
                    wt = jnp.broadcast_to(wv[tt], (_LANES,))
                    for s in range(_D // _LANES):
                        accs[s] = accs[s] + wt * emb_v[j, t, pl.ds(s * _LANES, _LANES)]
            for s in range(_D // _LANES):
                out_v[j, pl.ds(s * _LANES, _LANES)] = accs[s] * scale

        pltpu.sync_copy(out_v, out_hbm.at[pl.ds(base + c * _CH, _CH)])


def kernel(entity_id, entity_tokens, entity_counts, word_embeds, idf):
    b, l = entity_id.shape
    q = b * l
    qpw = q // _NW
    flat_ids = entity_id.reshape(q).astype(jnp.int32)

    mesh = plsc.VectorSubcoreMesh(core_axis_name="c", subcore_axis_name="s")
    run = functools.partial(
        pl.kernel,
        out_type=jax.ShapeDtypeStruct((q, _D), jnp.float32),
        mesh=mesh,
        compiler_params=pltpu.CompilerParams(
            needs_layout_passes=False, use_tc_tiling_on_sc=False),
        scratch_types=[
            pltpu.VMEM((_NUM_WORDS,), jnp.float32),
            pltpu.VMEM((qpw,), jnp.int32),
            pltpu.VMEM((_CH, _T), jnp.int32),
            pltpu.VMEM((_CH, _T), jnp.float32),
            pltpu.VMEM((_CH, _T, _D), jnp.float32),
            pltpu.VMEM((_CH, _D), jnp.float32),
            pltpu.SemaphoreType.DMA,
            pltpu.SemaphoreType.DMA,
        ],
    )(_body)
    out = run(flat_ids, entity_tokens, entity_counts, word_embeds, idf)
    return out.reshape(b, l, _D)


# speedup vs baseline: 28.5985x; 1.1094x over previous
import functools

import jax
import jax.numpy as jnp
from jax import lax
from jax.experimental import pallas as pl
from jax.experimental.pallas import tpu as pltpu
from jax.experimental.pallas import tpu_sc as plsc

_NUM_WORDS = 100000
_T = 32
_D = 64
_LANES = 16

_NC = 2
_NS = 16
_NW = _NC * _NS

_CH = 8
_HF = 4


def _body(ids_hbm, toks_hbm, cnts_hbm, emb_hbm, idf_hbm, out_hbm,
          idf_v, ids_v, toks2, cnts2, emb2, out_v, sem_t, gsem_a, gsem_b):
    qpw = ids_v.shape[0] - 2 * _CH
    n_chunks = qpw // _CH
    wid = lax.axis_index("s") * _NC + lax.axis_index("c")
    base = wid * qpw

    pltpu.sync_copy(idf_hbm, idf_v)
    ids_v[pl.ds(qpw, 2 * _CH)] = jnp.zeros((2 * _CH,), jnp.int32)
    pltpu.sync_copy(ids_hbm.at[pl.ds(base, qpw)], ids_v.at[pl.ds(0, qpw)])

    def fire_toks(c, par):
        idx = ids_v.at[pl.ds(c * _CH, _CH)]
        cp_t = pltpu.async_copy(toks_hbm.at[idx], toks2.at[par], sem_t)
        cp_c = pltpu.async_copy(cnts_hbm.at[idx], cnts2.at[par], sem_t)
        return cp_t, cp_c

    def fire_emb(par, qoff, slot, gsem):
        for jj in range(_HF):
            pltpu.async_copy(emb_hbm.at[toks2.at[par, qoff + jj]],
                             emb2.at[slot, jj], gsem)

    def drain_emb(slot, gsem):
        for jj in range(_HF):
            pltpu.make_async_copy(emb_hbm.at[pl.ds(0, _T)],
                                  emb2.at[slot, jj], gsem).wait()

    def compute4(par, qoff, slot):
        for jj in range(_HF):
            q = qoff + jj
            t0 = toks2[par, q, pl.ds(0, _LANES)]
            t1 = toks2[par, q, pl.ds(_LANES, _LANES)]
            w0 = cnts2[par, q, pl.ds(0, _LANES)] * plsc.load_gather(idf_v, [t0])
            w1 = cnts2[par, q, pl.ds(_LANES, _LANES)] * plsc.load_gather(idf_v, [t1])
            denom = jnp.maximum(jnp.sum(w0 + w1), 1e-6)
            scale = 1.0 / jnp.broadcast_to(denom, (_LANES,))

            accs = [jnp.zeros((_LANES,), jnp.float32) for _ in range(_D // _LANES)]
            for half, wv in enumerate((w0, w1)):
                for tt in range(_LANES):
                    t = half * _LANES + tt
                    wt = jnp.broadcast_to(wv[tt], (_LANES,))
                    for s in range(_D // _LANES):
                        accs[s] = accs[s] + wt * emb2[slot, jj, t, pl.ds(s * _LANES, _LANES)]
            for s in range(_D // _LANES):
                out_v[q, pl.ds(s * _LANES, _LANES)] = accs[s] * scale

    cp_t, cp_c = fire_toks(0, 0)
    cp_t.wait()
    cp_c.wait()
    fire_emb(0, 0, 0, gsem_a)

    @pl.loop(0, n_chunks // 2)
    def _pair(cp):
        for p in range(2):
            c = cp * 2 + p
            cp_t, cp_c = fire_toks(c + 1, 1 - p)
            fire_emb(p, _HF, 1, gsem_b)
            drain_emb(0, gsem_a)
            compute4(p, 0, 0)
            cp_t.wait()
            cp_c.wait()
            fire_emb(1 - p, 0, 0, gsem_a)
            drain_emb(1, gsem_b)
            compute4(p, _HF, 1)
            pltpu.sync_copy(out_v, out_hbm.at[pl.ds(base + c * _CH, _CH)])

    drain_emb(0, gsem_a)


def kernel(entity_id, entity_tokens, entity_counts, word_embeds, idf):
    b, l = entity_id.shape
    q = b * l
    qpw = q // _NW
    flat_ids = entity_id.reshape(q).astype(jnp.int32)

    mesh = plsc.VectorSubcoreMesh(core_axis_name="c", subcore_axis_name="s")
    run = functools.partial(
        pl.kernel,
        out_type=jax.ShapeDtypeStruct((q, _D), jnp.float32),
        mesh=mesh,
        compiler_params=pltpu.CompilerParams(
            needs_layout_passes=False, use_tc_tiling_on_sc=False),
        scratch_types=[
            pltpu.VMEM((_NUM_WORDS,), jnp.float32),
            pltpu.VMEM((qpw + 2 * _CH,), jnp.int32),
            pltpu.VMEM((2, _CH, _T), jnp.int32),
            pltpu.VMEM((2, _CH, _T), jnp.float32),
            pltpu.VMEM((2, _HF, _T, _D), jnp.float32),
            pltpu.VMEM((_CH, _D), jnp.float32),
            pltpu.SemaphoreType.DMA,
            pltpu.SemaphoreType.DMA,
            pltpu.SemaphoreType.DMA,
        ],
    )(_body)
    out = run(flat_ids, entity_tokens, entity_counts, word_embeds, idf)
    return out.reshape(b, l, _D)


# speedup vs baseline: 28.6534x; 1.0019x over previous
import functools

import jax
import jax.numpy as jnp
from jax import lax
from jax.experimental import pallas as pl
from jax.experimental.pallas import tpu as pltpu
from jax.experimental.pallas import tpu_sc as plsc

_NUM_WORDS = 100000
_T = 32
_D = 64
_LANES = 16

_NC = 2
_NS = 16
_NW = _NC * _NS

_CH = 8
_HF = 4
_HT = _HF * _T


def _body(ids_hbm, toks_hbm, cnts_hbm, emb_hbm, idf_hbm, out_hbm,
          idf_v, ids_v, toks2, cnts2, idxf, emb2, out_v, sem_t, gsem_a, gsem_b):
    qpw = ids_v.shape[0] - 2 * _CH
    n_chunks = qpw // _CH
    wid = lax.axis_index("s") * _NC + lax.axis_index("c")
    base = wid * qpw

    pltpu.sync_copy(idf_hbm, idf_v)
    ids_v[pl.ds(qpw, 2 * _CH)] = jnp.zeros((2 * _CH,), jnp.int32)
    pltpu.sync_copy(ids_hbm.at[pl.ds(base, qpw)], ids_v.at[pl.ds(0, qpw)])

    def fire_toks(c, par):
        idx = ids_v.at[pl.ds(c * _CH, _CH)]
        cp_t = pltpu.async_copy(toks_hbm.at[idx], toks2.at[par], sem_t)
        cp_c = pltpu.async_copy(cnts_hbm.at[idx], cnts2.at[par], sem_t)
        return cp_t, cp_c

    def fire_emb(par, hoff, slot, gsem):
        for jj in range(_HF):
            for h in range(2):
                idxf[par, hoff, pl.ds(jj * _T + h * _LANES, _LANES)] = \
                    toks2[par, hoff * _HF + jj, pl.ds(h * _LANES, _LANES)]
        pltpu.async_copy(emb_hbm.at[idxf.at[par, hoff]], emb2.at[slot], gsem)

    def drain_emb(slot, gsem):
        pltpu.make_async_copy(emb_hbm.at[pl.ds(0, _HT)],
                              emb2.at[slot], gsem).wait()

    def compute4(par, qoff, slot):
        for jj in range(_HF):
            q = qoff + jj
            t0 = toks2[par, q, pl.ds(0, _LANES)]
            t1 = toks2[par, q, pl.ds(_LANES, _LANES)]
            w0 = cnts2[par, q, pl.ds(0, _LANES)] * plsc.load_gather(idf_v, [t0])
            w1 = cnts2[par, q, pl.ds(_LANES, _LANES)] * plsc.load_gather(idf_v, [t1])
            denom = jnp.maximum(jnp.sum(w0 + w1), 1e-6)
            scale = 1.0 / jnp.broadcast_to(denom, (_LANES,))

            accs = [jnp.zeros((_LANES,), jnp.float32) for _ in range(_D // _LANES)]
            for half, wv in enumerate((w0, w1)):
                for tt in range(_LANES):
                    t = half * _LANES + tt
                    wt = jnp.broadcast_to(wv[tt], (_LANES,))
                    for s in range(_D // _LANES):
                        accs[s] = accs[s] + wt * emb2[slot, jj * _T + t, pl.ds(s * _LANES, _LANES)]
            for s in range(_D // _LANES):
                out_v[q, pl.ds(s * _LANES, _LANES)] = accs[s] * scale

    cp_t, cp_c = fire_toks(0, 0)
    cp_t.wait()
    cp_c.wait()
    fire_emb(0, 0, 0, gsem_a)

    @pl.loop(0, n_chunks // 2)
    def _pair(cp):
        for p in range(2):
            c = cp * 2 + p
            cp_t, cp_c = fire_toks(c + 1, 1 - p)
            fire_emb(p, 1, 1, gsem_b)
            drain_emb(0, gsem_a)
            compute4(p, 0, 0)
            cp_t.wait()
            cp_c.wait()
            fire_emb(1 - p, 0, 0, gsem_a)
            drain_emb(1, gsem_b)
            compute4(p, _HF, 1)
            pltpu.sync_copy(out_v, out_hbm.at[pl.ds(base + c * _CH, _CH)])

    drain_emb(0, gsem_a)


def kernel(entity_id, entity_tokens, entity_counts, word_embeds, idf):
    b, l = entity_id.shape
    q = b * l
    qpw = q // _NW
    flat_ids = entity_id.reshape(q).astype(jnp.int32)

    mesh = plsc.VectorSubcoreMesh(core_axis_name="c", subcore_axis_name="s")
    run = functools.partial(
        pl.kernel,
        out_type=jax.ShapeDtypeStruct((q, _D), jnp.float32),
        mesh=mesh,
        compiler_params=pltpu.CompilerParams(
            needs_layout_passes=False, use_tc_tiling_on_sc=False),
        scratch_types=[
            pltpu.VMEM((_NUM_WORDS,), jnp.float32),
            pltpu.VMEM((qpw + 2 * _CH,), jnp.int32),
            pltpu.VMEM((2, _CH, _T), jnp.int32),
            pltpu.VMEM((2, _CH, _T), jnp.float32),
            pltpu.VMEM((2, 2, _HT), jnp.int32),
            pltpu.VMEM((2, _HT, _D), jnp.float32),
            pltpu.VMEM((_CH, _D), jnp.float32),
            pltpu.SemaphoreType.DMA,
            pltpu.SemaphoreType.DMA,
            pltpu.SemaphoreType.DMA,
        ],
    )(_body)
    out = run(flat_ids, entity_tokens, entity_counts, word_embeds, idf)
    return out.reshape(b, l, _D)
